# baseline (speedup 1.0000x reference)
import functools
import jax
import jax.numpy as jnp
from jax.experimental import pallas as pl
from jax.experimental.pallas import tpu as pltpu


def _heads_body(z_ref, wo_ref, bo_ref, wb_ref, bb_ref, o_ref, b_ref):
    z = z_ref[...]
    o_ref[...] = jax.nn.sigmoid(z @ wo_ref[...] + bo_ref[0])
    b_ref[...] = jax.nn.sigmoid(z @ wb_ref[...] + bb_ref[0])


def _heads(z, Wo, bo, Wb, bb):
    g = z.shape[0]
    return pl.pallas_call(
        _heads_body,
        out_shape=[
            jax.ShapeDtypeStruct((g, 1), jnp.float32),
            jax.ShapeDtypeStruct((g, 1), jnp.float32),
        ],
    )(z, Wo, bo, Wb, bb)


def _gcn_conv(x, src, dst, norm, W, b, n):
    h = x @ W
    msg = h[src] * norm[:, None]
    out = jnp.zeros((n, h.shape[1]), dtype=x.dtype).at[dst].add(msg)
    return out + b


def _bn_relu(x, g, be):
    mu = jnp.mean(x, axis=0)
    var = jnp.var(x, axis=0)
    return jax.nn.relu((x - mu) * jax.lax.rsqrt(var + 1e-5) * g + be)


def kernel(x, edge_index, edge_weight, batch, global_features,
           W1, b1, g1, be1, W2, b2, g2, be2, Wo, bo, Wb, bb):
    n = x.shape[0]
    src = edge_index[0]
    dst = edge_index[1]
    deg = jnp.zeros((n,), jnp.float32).at[dst].add(edge_weight) + 1.0
    dinv = jax.lax.rsqrt(deg)
    norm = dinv[src] * edge_weight * dinv[dst]
    dinv2 = dinv * dinv

    h = x @ W1
    s1 = jnp.zeros((n, h.shape[1]), jnp.float32).at[dst].add(h[src] * norm[:, None])
    a1 = s1 + h * dinv2[:, None] + b1
    h1 = _bn_relu(a1, g1, be1)

    h2 = h1 @ W2
    s2 = jnp.zeros((n, h2.shape[1]), jnp.float32).at[dst].add(h2[src] * norm[:, None])
    a2 = s2 + h2 * dinv2[:, None] + b2
    hf = _bn_relu(a2, g2, be2)

    gnum = global_features.shape[0]
    sums = jax.ops.segment_sum(hf, batch, num_segments=gnum)
    cnt = jax.ops.segment_sum(jnp.ones((n,), jnp.float32), batch, num_segments=gnum)
    pooled = sums / jnp.maximum(cnt, 1.0)[:, None]
    z = jnp.concatenate([pooled, global_features], axis=1)
    orange, blue = _heads(z, Wo, bo, Wb, bb)
    return (orange, blue)


# speedup vs baseline: 2.7608x; 2.7608x over previous
import functools
import jax
import jax.numpy as jnp
from jax import lax
from jax.experimental import pallas as pl
from jax.experimental.pallas import tpu as pltpu
from jax.experimental.pallas import tpu_sc as plsc

_N = 100000
_E = 3200000
_H = 32
_HH = 16
_R = _E // 128
_ROWS_PER_W = _R // 32
_NPAD = 16 * 6256
_NH = _N // 2
_G = 1024
_GP = 1040

_mesh = plsc.VectorSubcoreMesh(core_axis_name="c", subcore_axis_name="s")
_sc_params = pltpu.CompilerParams(
    needs_layout_passes=False, use_tc_tiling_on_sc=False)


def _deg_body(dst_hbm, w_hbm, out_hbm, dstv, wv, zbuf, degs):
    c = lax.axis_index("c")
    s = lax.axis_index("s")
    wid = c * 16 + s

    zv = jnp.zeros((16,), jnp.float32)

    def zb(i, _):
        zbuf[pl.ds(i * 16, 16)] = zv
        return 0

    lax.fori_loop(0, 391, zb, 0)
    pltpu.sync_copy(zbuf, degs.at[pl.ds(s * 6256, 6256)])
    plsc.subcore_barrier()

    def row_body(i, _):
        r = wid * _ROWS_PER_W + i
        pltpu.sync_copy(dst_hbm.at[pl.ds(r, 1), :], dstv)
        pltpu.sync_copy(w_hbm.at[pl.ds(r, 1), :], wv)
        pltpu.sync_copy(wv.at[0], degs.at[dstv.at[0]], add=True)
        return 0

    lax.fori_loop(0, _ROWS_PER_W, row_body, 0)

    @pl.when(wid < 8)
    def _extra():
        r = 32 * _ROWS_PER_W + wid
        pltpu.sync_copy(dst_hbm.at[pl.ds(r, 1), :], dstv)
        pltpu.sync_copy(w_hbm.at[pl.ds(r, 1), :], wv)
        pltpu.sync_copy(wv.at[0], degs.at[dstv.at[0]], add=True)

    plsc.subcore_barrier()
    pltpu.sync_copy(degs.at[pl.ds(s * 6256, 6256)], zbuf)
    pltpu.sync_copy(zbuf, out_hbm.at[pl.ds(c * _NPAD + s * 6256, 6256)])


def _deg_sc(dst2d, w2d):
    f = pl.kernel(
        _deg_body,
        out_type=jax.ShapeDtypeStruct((2 * _NPAD,), jnp.float32),
        mesh=_mesh,
        compiler_params=pltpu.CompilerParams(needs_layout_passes=False),
        scratch_types=[
            pltpu.VMEM((1, 128), jnp.int32),
            pltpu.VMEM((1, 128), jnp.float32),
            pltpu.VMEM((6256,), jnp.float32),
            pltpu.VMEM_SHARED((_NPAD,), jnp.float32),
        ],
    )
    return f(dst2d, w2d)


_NQ = _N // 4


def _mp_body(hsa_hbm, hsb_hbm, ia_hbm, ib_hbm, srcf, dstf, wf, dinv_hbm,
             outa_hbm, outb_hbm, dinv_v, rows, srcv, dstv, wv, idxv, wsv, acc):
    c = lax.axis_index("c")
    s = lax.axis_index("s")

    def tile_rows(src_at, dst_at):
        @pl.when(s < 15)
        def _():
            for k in range(12):
                pltpu.sync_copy(src_at(k * 128, 128), rows)
                pltpu.sync_copy(rows, dst_at(k * 128, 128))
            pltpu.sync_copy(src_at(1536, 24), rows.at[pl.ds(0, 24), :])
            pltpu.sync_copy(rows.at[pl.ds(0, 24), :], dst_at(1536, 24))

        @pl.when(s == 15)
        def _():
            for k in range(12):
                pltpu.sync_copy(src_at(k * 128, 128), rows)
                pltpu.sync_copy(rows, dst_at(k * 128, 128))
            pltpu.sync_copy(src_at(1536, 64), rows.at[pl.ds(0, 64), :])
            pltpu.sync_copy(rows.at[pl.ds(0, 64), :], dst_at(1536, 64))

    pltpu.sync_copy(dinv_hbm, dinv_v)

    start = s * 1562 + jnp.minimum(s, 8)
    nrows = jnp.where(s < 8, 1563, 1562)
    tstart = s * 1560

    for hs_hbm, init_hbm, out_hbm in ((hsa_hbm, ia_hbm, outa_hbm),
                                      (hsb_hbm, ib_hbm, outb_hbm)):
        for q in range(2):
            lo = c * _NH + q * _NQ

            tile_rows(lambda o, n: init_hbm.at[pl.ds(lo + tstart + o, n), :],
                      lambda o, n: acc.at[pl.ds(tstart + o, n), :])
            plsc.subcore_barrier()

            def row_body(i, _):
                off = (start + i) * 128
                pltpu.sync_copy(srcf.at[pl.ds(off, 128)], srcv)
                pltpu.sync_copy(dstf.at[pl.ds(off, 128)], dstv)
                pltpu.sync_copy(wf.at[pl.ds(off, 128)], wv)
                pltpu.sync_copy(hs_hbm.at[srcv], rows)
                for g in range(8):
                    d16 = dstv[pl.ds(g * 16, 16)]
                    w16 = wv[pl.ds(g * 16, 16)]
                    ws = w16 * plsc.load_gather(dinv_v, [d16])
                    wsv[pl.ds(g * 16, 16)] = ws
                    local = d16 - lo
                    m = (local >= 0) & (local < _NQ)
                    idxv[pl.ds(g * 16, 16)] = jnp.where(m, local, _NQ)
                for e in range(128):
                    sp = plsc.load_gather(wsv, [jnp.full((16,), e, jnp.int32)])
                    rows[e, pl.ds(0, _HH)] = rows[e, pl.ds(0, _HH)] * sp
                pltpu.sync_copy(rows, acc.at[idxv], add=True)
                return 0

            lax.fori_loop(0, nrows, row_body, 0)
            plsc.subcore_barrier()

            tile_rows(lambda o, n: acc.at[pl.ds(tstart + o, n), :],
                      lambda o, n: out_hbm.at[pl.ds(lo + tstart + o, n), :])
            plsc.subcore_barrier()


_mp_kernel = pl.kernel(
    _mp_body,
    out_type=[
        jax.ShapeDtypeStruct((_NPAD, _HH), jnp.float32),
        jax.ShapeDtypeStruct((_NPAD, _HH), jnp.float32),
    ],
    mesh=_mesh,
    compiler_params=_sc_params,
    scratch_types=[
        pltpu.VMEM((_NPAD,), jnp.float32),
        pltpu.VMEM((128, _HH), jnp.float32),
        pltpu.VMEM((128,), jnp.int32),
        pltpu.VMEM((128,), jnp.int32),
        pltpu.VMEM((128,), jnp.float32),
        pltpu.VMEM((128,), jnp.int32),
        pltpu.VMEM((128,), jnp.float32),
        pltpu.VMEM_SHARED((_NQ + 8, _HH), jnp.float32),
    ],
)


def _pool_body(sa_hbm, sb_hbm, ss_hbm, batchf, sa_out, sb_out, cnt_out,
               hfa, hfb, hea, heb, bv, onesv, zb1, ssv,
               spa, spb, cnt_sp):
    c = lax.axis_index("c")
    s = lax.axis_index("s")
    wid = c * 16 + s

    zv = jnp.zeros((16,), jnp.float32)
    ov = jnp.ones((16,), jnp.float32)
    for k in range(8):
        onesv[pl.ds(k * 16, 16)] = ov

    def zrow(i, _):
        hea[i, pl.ds(0, 16)] = zv
        return 0

    lax.fori_loop(0, 128, zrow, 0)

    def zb(i, _):
        zb1[pl.ds(i * 16, 16)] = zv
        return 0

    lax.fori_loop(0, 65, zb, 0)

    @pl.when(s == 0)
    def _():
        pltpu.sync_copy(zb1, cnt_sp)

    @pl.when(s < 13)
    def _():
        pltpu.sync_copy(hea.at[pl.ds(0, 80), :], spa.at[pl.ds(s * 80, 80), :])
        pltpu.sync_copy(hea.at[pl.ds(0, 80), :], spb.at[pl.ds(s * 80, 80), :])

    pltpu.sync_copy(ss_hbm, ssv)
    plsc.subcore_barrier()

    sca = ssv[0, pl.ds(0, 16)]
    scb = ssv[0, pl.ds(16, 16)]
    sha = ssv[1, pl.ds(0, 16)]
    shb = ssv[1, pl.ds(16, 16)]
    start = wid * 24 + jnp.minimum(wid, 14)

    def bn_rows(refa, refb, n):
        def body(r, _):
            refa[r, pl.ds(0, 16)] = jnp.maximum(
                refa[r, pl.ds(0, 16)] * sca + sha, 0.0)
            refb[r, pl.ds(0, 16)] = jnp.maximum(
                refb[r, pl.ds(0, 16)] * scb + shb, 0.0)
            return 0
        lax.fori_loop(0, n, body, 0)

    pltpu.sync_copy(sa_hbm.at[pl.ds(start * 128, 3072), :], hfa)
    pltpu.sync_copy(sb_hbm.at[pl.ds(start * 128, 3072), :], hfb)
    bn_rows(hfa, hfb, 3072)
    for j in range(24):
        pltpu.sync_copy(batchf.at[pl.ds((start + j) * 128, 128)], bv)
        pltpu.sync_copy(hfa.at[pl.ds(j * 128, 128), :], spa.at[bv], add=True)
        pltpu.sync_copy(hfb.at[pl.ds(j * 128, 128), :], spb.at[bv], add=True)
        pltpu.sync_copy(onesv, cnt_sp.at[bv], add=True)

    @pl.when(wid < 14)
    def _():
        pltpu.sync_copy(sa_hbm.at[pl.ds((start + 24) * 128, 128), :], hea)
        pltpu.sync_copy(sb_hbm.at[pl.ds((start + 24) * 128, 128), :], heb)
        bn_rows(hea, heb, 128)
        pltpu.sync_copy(batchf.at[pl.ds((start + 24) * 128, 128)], bv)
        pltpu.sync_copy(hea, spa.at[bv], add=True)
        pltpu.sync_copy(heb, spb.at[bv], add=True)
        pltpu.sync_copy(onesv, cnt_sp.at[bv], add=True)

    plsc.subcore_barrier()

    @pl.when(s < 13)
    def _():
        pltpu.sync_copy(spa.at[pl.ds(s * 80, 80), :], hea.at[pl.ds(0, 80), :])
        pltpu.sync_copy(hea.at[pl.ds(0, 80), :],
                        sa_out.at[pl.ds(c * _GP + s * 80, 80), :])
        pltpu.sync_copy(spb.at[pl.ds(s * 80, 80), :], heb.at[pl.ds(0, 80), :])
        pltpu.sync_copy(heb.at[pl.ds(0, 80), :],
                        sb_out.at[pl.ds(c * _GP + s * 80, 80), :])

    @pl.when(s == 0)
    def _():
        pltpu.sync_copy(cnt_sp, zb1)
        pltpu.sync_copy(zb1, cnt_out.at[pl.ds(c * _GP, _GP)])


def _pool_sc(sa, sb, ss2, batchf):
    f = pl.kernel(
        _pool_body,
        out_type=[
            jax.ShapeDtypeStruct((2 * _GP, _HH), jnp.float32),
            jax.ShapeDtypeStruct((2 * _GP, _HH), jnp.float32),
            jax.ShapeDtypeStruct((2 * _GP,), jnp.float32),
        ],
        mesh=_mesh,
        compiler_params=_sc_params,
        scratch_types=[
            pltpu.VMEM((3072, _HH), jnp.float32),
            pltpu.VMEM((3072, _HH), jnp.float32),
            pltpu.VMEM((128, _HH), jnp.float32),
            pltpu.VMEM((128, _HH), jnp.float32),
            pltpu.VMEM((128,), jnp.int32),
            pltpu.VMEM((128,), jnp.float32),
            pltpu.VMEM((_GP,), jnp.float32),
            pltpu.VMEM((2, _H), jnp.float32),
            pltpu.VMEM_SHARED((_GP, _HH), jnp.float32),
            pltpu.VMEM_SHARED((_GP, _HH), jnp.float32),
            pltpu.VMEM_SHARED((_GP,), jnp.float32),
        ],
    )
    return f(sa, sb, ss2, batchf)


_BR = 2000
_NB = _N // _BR


def _prep_body(x_ref, p0_ref, p1_ref, w1_ref, b1_ref,
               dinv_ref, hsa_ref, hsb_ref, ia_ref, ib_ref):
    deg = p0_ref[...] + p1_ref[...] + 1.0
    dinv = lax.rsqrt(deg)
    h = jnp.dot(x_ref[...], w1_ref[...], preferred_element_type=jnp.float32)
    hs = h * dinv
    init = hs * dinv + b1_ref[...]
    dinv_ref[...] = dinv
    hsa_ref[...] = hs[:, :_HH]
    hsb_ref[...] = hs[:, _HH:]
    ia_ref[...] = init[:, :_HH]
    ib_ref[...] = init[:, _HH:]


def _prep_tc(x, p0, p1, W1, b1):
    return pl.pallas_call(
        _prep_body,
        grid=(_NB,),
        in_specs=[
            pl.BlockSpec((_BR, 13), lambda i: (i, 0)),
            pl.BlockSpec((_BR, 1), lambda i: (i, 0)),
            pl.BlockSpec((_BR, 1), lambda i: (i, 0)),
            pl.BlockSpec((13, _H), lambda i: (0, 0)),
            pl.BlockSpec((_H,), lambda i: (0,)),
        ],
        out_specs=[
            pl.BlockSpec((_BR, 1), lambda i: (i, 0)),
            pl.BlockSpec((_BR, _HH), lambda i: (i, 0)),
            pl.BlockSpec((_BR, _HH), lambda i: (i, 0)),
            pl.BlockSpec((_BR, _HH), lambda i: (i, 0)),
            pl.BlockSpec((_BR, _HH), lambda i: (i, 0)),
        ],
        out_shape=[
            jax.ShapeDtypeStruct((_N, 1), jnp.float32),
            jax.ShapeDtypeStruct((_N, _HH), jnp.float32),
            jax.ShapeDtypeStruct((_N, _HH), jnp.float32),
            jax.ShapeDtypeStruct((_N, _HH), jnp.float32),
            jax.ShapeDtypeStruct((_N, _HH), jnp.float32),
        ],
    )(x, p0, p1, W1, b1)


def _stats_body(xa_ref, xb_ref, g_ref, be_ref, o_ref, acc):
    i = pl.program_id(0)

    @pl.when(i == 0)
    def _():
        acc[...] = jnp.zeros_like(acc)

    xb = jnp.concatenate([xa_ref[...], xb_ref[...]], axis=1)
    acc[...] += jnp.stack([jnp.sum(xb, 0), jnp.sum(xb * xb, 0)])

    @pl.when(i == _NB - 1)
    def _():
        mu = acc[0, :] * (1.0 / _N)
        var = acc[1, :] * (1.0 / _N) - mu * mu
        sc = g_ref[...] * lax.rsqrt(var + 1e-5)
        o_ref[...] = jnp.stack([sc, be_ref[...] - mu * sc])


def _stats_tc(Sa, Sb, g, be):
    return pl.pallas_call(
        _stats_body,
        grid=(_NB,),
        in_specs=[
            pl.BlockSpec((_BR, _HH), lambda i: (i, 0)),
            pl.BlockSpec((_BR, _HH), lambda i: (i, 0)),
            pl.BlockSpec((_H,), lambda i: (0,)),
            pl.BlockSpec((_H,), lambda i: (0,)),
        ],
        out_specs=pl.BlockSpec((2, _H), lambda i: (0, 0)),
        out_shape=jax.ShapeDtypeStruct((2, _H), jnp.float32),
        scratch_shapes=[pltpu.VMEM((2, _H), jnp.float32)],
    )(Sa, Sb, g, be)


def _apply_body(sa_ref, sb_ref, ss_ref, dinv_ref, w2_ref, b2_ref,
                hsa_ref, hsb_ref, ia_ref, ib_ref):
    xb = jnp.concatenate([sa_ref[...], sb_ref[...]], axis=1)
    y = jnp.maximum(xb * ss_ref[0, :] + ss_ref[1, :], 0.0)
    hb = jnp.dot(y, w2_ref[...], preferred_element_type=jnp.float32)
    dinv = dinv_ref[...]
    hs = hb * dinv
    init = hs * dinv + b2_ref[...]
    hsa_ref[...] = hs[:, :_HH]
    hsb_ref[...] = hs[:, _HH:]
    ia_ref[...] = init[:, :_HH]
    ib_ref[...] = init[:, _HH:]


def _apply_tc(S1a, S1b, ss1, dinv, W2, b2):
    return pl.pallas_call(
        _apply_body,
        grid=(_NB,),
        in_specs=[
            pl.BlockSpec((_BR, _HH), lambda i: (i, 0)),
            pl.BlockSpec((_BR, _HH), lambda i: (i, 0)),
            pl.BlockSpec((2, _H), lambda i: (0, 0)),
            pl.BlockSpec((_BR, 1), lambda i: (i, 0)),
            pl.BlockSpec((_H, _H), lambda i: (0, 0)),
            pl.BlockSpec((_H,), lambda i: (0,)),
        ],
        out_specs=[
            pl.BlockSpec((_BR, _HH), lambda i: (i, 0)),
            pl.BlockSpec((_BR, _HH), lambda i: (i, 0)),
            pl.BlockSpec((_BR, _HH), lambda i: (i, 0)),
            pl.BlockSpec((_BR, _HH), lambda i: (i, 0)),
        ],
        out_shape=[
            jax.ShapeDtypeStruct((_N, _HH), jnp.float32),
            jax.ShapeDtypeStruct((_N, _HH), jnp.float32),
            jax.ShapeDtypeStruct((_N, _HH), jnp.float32),
            jax.ShapeDtypeStruct((_N, _HH), jnp.float32),
        ],
    )(S1a, S1b, ss1, dinv, W2, b2)


def _heads_body(sa_ref, sb_ref, cn_ref, gf_ref, wo_ref, bo_ref,
                wb_ref, bb_ref, o_ref, b_ref):
    sa = sa_ref[0, :_G, :] + sa_ref[1, :_G, :]
    sb = sb_ref[0, :_G, :] + sb_ref[1, :_G, :]
    cnt = jnp.maximum(cn_ref[0, :_G] + cn_ref[1, :_G], 1.0)[:, None]
    pa = sa / cnt
    pb = sb / cnt
    gf = gf_ref[...]
    o_ref[...] = jax.nn.sigmoid(
        pa @ wo_ref[:_HH, :] + pb @ wo_ref[_HH:_H, :]
        + gf @ wo_ref[_H:, :] + bo_ref[0])
    b_ref[...] = jax.nn.sigmoid(
        pa @ wb_ref[:_HH, :] + pb @ wb_ref[_HH:_H, :]
        + gf @ wb_ref[_H:, :] + bb_ref[0])


def _heads_tc(sa, sb, cnt, gf, Wo, bo, Wb, bb):
    return pl.pallas_call(
        _heads_body,
        out_shape=[
            jax.ShapeDtypeStruct((_G, 1), jnp.float32),
            jax.ShapeDtypeStruct((_G, 1), jnp.float32),
        ],
    )(sa, sb, cnt, gf, Wo, bo, Wb, bb)


def kernel(x, edge_index, edge_weight, batch, global_features,
           W1, b1, g1, be1, W2, b2, g2, be2, Wo, bo, Wb, bb):
    srcf = edge_index[0]
    dstf = edge_index[1]
    degp = _deg_sc(dstf.reshape(_R, 128), edge_weight.reshape(_R, 128))
    p0 = degp[:_N].reshape(_N, 1)
    p1 = degp[_NPAD:_NPAD + _N].reshape(_N, 1)

    dinv, hs1a, hs1b, i1a, i1b = _prep_tc(x, p0, p1, W1, b1)
    dinvf = jnp.pad(dinv[:, 0], (0, _NPAD - _N))

    S1a, S1b = _mp_kernel(hs1a, hs1b, i1a, i1b, srcf, dstf, edge_weight, dinvf)
    ss1 = _stats_tc(S1a[:_N], S1b[:_N], g1, be1)
    hs2a, hs2b, i2a, i2b = _apply_tc(S1a[:_N], S1b[:_N], ss1, dinv, W2, b2)

    S2a, S2b = _mp_kernel(hs2a, hs2b, i2a, i2b, srcf, dstf, edge_weight, dinvf)
    ss2 = _stats_tc(S2a[:_N], S2b[:_N], g2, be2)

    batchf = jnp.pad(batch, (0, _NPAD - _N), constant_values=_G)
    sa, sb, cnt = _pool_sc(S2a, S2b, ss2, batchf)
    orange, blue = _heads_tc(sa.reshape(2, _GP, _HH), sb.reshape(2, _GP, _HH),
                             cnt.reshape(2, _GP), global_features,
                             Wo, bo, Wb, bb)
    return (orange, blue)


# speedup vs baseline: 5.1319x; 1.8588x over previous
import functools
import jax
import jax.numpy as jnp
from jax import lax
from jax.experimental import pallas as pl
from jax.experimental.pallas import tpu as pltpu
from jax.experimental.pallas import tpu_sc as plsc

_N = 100000
_E = 3200000
_H = 32
_HH = 16
_R = _E // 128
_ROWS_PER_W = _R // 32
_NPAD = 16 * 6256
_NH = _N // 2
_G = 1024
_GP = 1040

_mesh = plsc.VectorSubcoreMesh(core_axis_name="c", subcore_axis_name="s")
_sc_params = pltpu.CompilerParams(
    needs_layout_passes=False, use_tc_tiling_on_sc=False)


def _deg_body(dst_hbm, w_hbm, out_hbm, dstv, wv, zbuf, degs):
    c = lax.axis_index("c")
    s = lax.axis_index("s")
    wid = c * 16 + s

    zv = jnp.zeros((16,), jnp.float32)

    def zb(i, _):
        zbuf[pl.ds(i * 16, 16)] = zv
        return 0

    lax.fori_loop(0, 391, zb, 0)
    pltpu.sync_copy(zbuf, degs.at[pl.ds(s * 6256, 6256)])
    plsc.subcore_barrier()

    def row_body(i, _):
        r = wid * _ROWS_PER_W + i
        pltpu.sync_copy(dst_hbm.at[pl.ds(r, 1), :], dstv)
        pltpu.sync_copy(w_hbm.at[pl.ds(r, 1), :], wv)
        pltpu.sync_copy(wv.at[0], degs.at[dstv.at[0]], add=True)
        return 0

    lax.fori_loop(0, _ROWS_PER_W, row_body, 0)

    @pl.when(wid < 8)
    def _extra():
        r = 32 * _ROWS_PER_W + wid
        pltpu.sync_copy(dst_hbm.at[pl.ds(r, 1), :], dstv)
        pltpu.sync_copy(w_hbm.at[pl.ds(r, 1), :], wv)
        pltpu.sync_copy(wv.at[0], degs.at[dstv.at[0]], add=True)

    plsc.subcore_barrier()
    pltpu.sync_copy(degs.at[pl.ds(s * 6256, 6256)], zbuf)
    pltpu.sync_copy(zbuf, out_hbm.at[pl.ds(c * _NPAD + s * 6256, 6256)])


def _deg_sc(dst2d, w2d):
    f = pl.kernel(
        _deg_body,
        out_type=jax.ShapeDtypeStruct((2 * _NPAD,), jnp.float32),
        mesh=_mesh,
        compiler_params=pltpu.CompilerParams(needs_layout_passes=False),
        scratch_types=[
            pltpu.VMEM((1, 128), jnp.int32),
            pltpu.VMEM((1, 128), jnp.float32),
            pltpu.VMEM((6256,), jnp.float32),
            pltpu.VMEM_SHARED((_NPAD,), jnp.float32),
        ],
    )
    return f(dst2d, w2d)


_NQ = _N // 4


def _mp_body(hsa_hbm, hsb_hbm, ia_hbm, ib_hbm, pk_hbm, dinv_hbm,
             outa_hbm, outb_hbm, dinv_v, rows, rows1, pe0, pe1,
             idx0, idx1, wsv, acc, ld0, ld1, g0, g1, sc0, sc1):
    c = lax.axis_index("c")
    s = lax.axis_index("s")

    def tile_rows(src_at, dst_at):
        @pl.when(s < 15)
        def _():
            for k in range(12):
                pltpu.sync_copy(src_at(k * 128, 128), rows)
                pltpu.sync_copy(rows, dst_at(k * 128, 128))
            pltpu.sync_copy(src_at(1536, 24), rows.at[pl.ds(0, 24), :])
            pltpu.sync_copy(rows.at[pl.ds(0, 24), :], dst_at(1536, 24))

        @pl.when(s == 15)
        def _():
            for k in range(12):
                pltpu.sync_copy(src_at(k * 128, 128), rows)
                pltpu.sync_copy(rows, dst_at(k * 128, 128))
            pltpu.sync_copy(src_at(1536, 64), rows.at[pl.ds(0, 64), :])
            pltpu.sync_copy(rows.at[pl.ds(0, 64), :], dst_at(1536, 64))

    pltpu.sync_copy(dinv_hbm, dinv_v)

    start = s * 1562 + jnp.minimum(s, 8)
    tstart = s * 1560

    def compute(pe, rws, idxv, lo):
        def cbody(g, _):
            d16 = pe[pl.ds(g * 16 + 128, 16)]
            w16 = plsc.bitcast(pe[pl.ds(g * 16 + 256, 16)], jnp.float32)
            ws = w16 * plsc.load_gather(dinv_v, [d16])
            wsv[pl.ds(g * 16, 16)] = ws
            local = d16 - lo
            m = (local >= 0) & (local < _NQ)
            idxv[pl.ds(g * 16, 16)] = jnp.where(m, local, _NQ)
            for el in range(16):
                e = g * 16 + el
                sp = plsc.load_gather(wsv, [jnp.full((16,), e, jnp.int32)])
                rws[e, pl.ds(0, _HH)] = rws[e, pl.ds(0, _HH)] * sp
            return 0

        lax.fori_loop(0, 8, cbody, 0)

    def ld_issue(r, pe, sem):
        pltpu.make_async_copy(pk_hbm.at[pl.ds(r * 384, 384)], pe, sem).start()

    def ld_wait(pe, sem):
        pltpu.make_async_copy(pk_hbm.at[pl.ds(0, 384)], pe, sem).wait()

    for hs_hbm, init_hbm, out_hbm in ((hsa_hbm, ia_hbm, outa_hbm),
                                      (hsb_hbm, ib_hbm, outb_hbm)):

        def g_issue(pe, rws, sem):
            pltpu.make_async_copy(hs_hbm.at[pe.at[pl.ds(0, 128)]], rws,
                                  sem).start()

        def g_wait(pe, rws, sem):
            pltpu.make_async_copy(hs_hbm.at[pe.at[pl.ds(0, 128)]], rws,
                                  sem).wait()

        def sc_issue(rws, idxv, sem):
            pltpu.make_async_copy(rws, acc.at[idxv], sem).start()

        def sc_wait(rws, idxv, sem):
            pltpu.make_async_copy(rws, acc.at[idxv], sem).wait()

        for q in range(2):
            lo = c * _NH + q * _NQ

            tile_rows(lambda o, n: init_hbm.at[pl.ds(lo + tstart + o, n), :],
                      lambda o, n: acc.at[pl.ds(tstart + o, n), :])
            plsc.subcore_barrier()

            nh = 781
            ld_issue(start, pe0, ld0)

            def pipe_body(i, _):
                r0 = start + 2 * i
                ld_wait(pe0, ld0)

                @pl.when(i > 0)
                def _():
                    sc_wait(rows, idx0, sc0)

                g_issue(pe0, rows, g0)
                ld_issue(r0 + 1, pe1, ld1)
                g_wait(pe0, rows, g0)
                compute(pe0, rows, idx0, lo)
                sc_issue(rows, idx0, sc0)

                ld_wait(pe1, ld1)

                @pl.when(i > 0)
                def _():
                    sc_wait(rows1, idx1, sc1)

                g_issue(pe1, rows1, g1)

                @pl.when(i < nh - 1)
                def _():
                    ld_issue(r0 + 2, pe0, ld0)

                g_wait(pe1, rows1, g1)
                compute(pe1, rows1, idx1, lo)
                sc_issue(rows1, idx1, sc1)
                return 0

            lax.fori_loop(0, nh, pipe_body, 0)
            sc_wait(rows, idx0, sc0)
            sc_wait(rows1, idx1, sc1)

            @pl.when(s < 8)
            def _():
                r = start + 1562
                pltpu.sync_copy(pk_hbm.at[pl.ds(r * 384, 384)], pe0)
                pltpu.sync_copy(hs_hbm.at[pe0.at[pl.ds(0, 128)]], rows)
                compute(pe0, rows, idx0, lo)
                pltpu.sync_copy(rows, acc.at[idx0], add=True)

            plsc.subcore_barrier()

            tile_rows(lambda o, n: acc.at[pl.ds(tstart + o, n), :],
                      lambda o, n: out_hbm.at[pl.ds(lo + tstart + o, n), :])
            plsc.subcore_barrier()


_mp_kernel = pl.kernel(
    _mp_body,
    out_type=[
        jax.ShapeDtypeStruct((_NPAD, _HH), jnp.float32),
        jax.ShapeDtypeStruct((_NPAD, _HH), jnp.float32),
    ],
    mesh=_mesh,
    compiler_params=_sc_params,
    scratch_types=[
        pltpu.VMEM((_NPAD,), jnp.float32),
        pltpu.VMEM((128, _HH), jnp.float32),
        pltpu.VMEM((128, _HH), jnp.float32),
        pltpu.VMEM((384,), jnp.int32),
        pltpu.VMEM((384,), jnp.int32),
        pltpu.VMEM((128,), jnp.int32),
        pltpu.VMEM((128,), jnp.int32),
        pltpu.VMEM((128,), jnp.float32),
        pltpu.VMEM_SHARED((_NQ + 8, _HH), jnp.float32),
        pltpu.SemaphoreType.DMA,
        pltpu.SemaphoreType.DMA,
        pltpu.SemaphoreType.DMA,
        pltpu.SemaphoreType.DMA,
        pltpu.SemaphoreType.DMA,
        pltpu.SemaphoreType.DMA,
    ],
)


def _pool_body(sa_hbm, sb_hbm, ss_hbm, batchf, sa_out, sb_out, cnt_out,
               hfa, hfb, hea, heb, bv, onesv, zb1, ssv,
               spa, spb, cnt_sp):
    c = lax.axis_index("c")
    s = lax.axis_index("s")
    wid = c * 16 + s

    zv = jnp.zeros((16,), jnp.float32)
    ov = jnp.ones((16,), jnp.float32)
    for k in range(8):
        onesv[pl.ds(k * 16, 16)] = ov

    def zrow(i, _):
        hea[i, pl.ds(0, 16)] = zv
        return 0

    lax.fori_loop(0, 128, zrow, 0)

    def zb(i, _):
        zb1[pl.ds(i * 16, 16)] = zv
        return 0

    lax.fori_loop(0, 65, zb, 0)

    @pl.when(s == 0)
    def _():
        pltpu.sync_copy(zb1, cnt_sp)

    @pl.when(s < 13)
    def _():
        pltpu.sync_copy(hea.at[pl.ds(0, 80), :], spa.at[pl.ds(s * 80, 80), :])
        pltpu.sync_copy(hea.at[pl.ds(0, 80), :], spb.at[pl.ds(s * 80, 80), :])

    pltpu.sync_copy(ss_hbm, ssv)
    plsc.subcore_barrier()

    sca = ssv[0, pl.ds(0, 16)]
    scb = ssv[0, pl.ds(16, 16)]
    sha = ssv[1, pl.ds(0, 16)]
    shb = ssv[1, pl.ds(16, 16)]
    start = wid * 24 + jnp.minimum(wid, 14)

    def bn_rows(refa, refb, n):
        def body(r, _):
            refa[r, pl.ds(0, 16)] = jnp.maximum(
                refa[r, pl.ds(0, 16)] * sca + sha, 0.0)
            refb[r, pl.ds(0, 16)] = jnp.maximum(
                refb[r, pl.ds(0, 16)] * scb + shb, 0.0)
            return 0
        lax.fori_loop(0, n, body, 0)

    pltpu.sync_copy(sa_hbm.at[pl.ds(start * 128, 3072), :], hfa)
    pltpu.sync_copy(sb_hbm.at[pl.ds(start * 128, 3072), :], hfb)
    bn_rows(hfa, hfb, 3072)
    for j in range(24):
        pltpu.sync_copy(batchf.at[pl.ds((start + j) * 128, 128)], bv)
        pltpu.sync_copy(hfa.at[pl.ds(j * 128, 128), :], spa.at[bv], add=True)
        pltpu.sync_copy(hfb.at[pl.ds(j * 128, 128), :], spb.at[bv], add=True)
        pltpu.sync_copy(onesv, cnt_sp.at[bv], add=True)

    @pl.when(wid < 14)
    def _():
        pltpu.sync_copy(sa_hbm.at[pl.ds((start + 24) * 128, 128), :], hea)
        pltpu.sync_copy(sb_hbm.at[pl.ds((start + 24) * 128, 128), :], heb)
        bn_rows(hea, heb, 128)
        pltpu.sync_copy(batchf.at[pl.ds((start + 24) * 128, 128)], bv)
        pltpu.sync_copy(hea, spa.at[bv], add=True)
        pltpu.sync_copy(heb, spb.at[bv], add=True)
        pltpu.sync_copy(onesv, cnt_sp.at[bv], add=True)

    plsc.subcore_barrier()

    @pl.when(s < 13)
    def _():
        pltpu.sync_copy(spa.at[pl.ds(s * 80, 80), :], hea.at[pl.ds(0, 80), :])
        pltpu.sync_copy(hea.at[pl.ds(0, 80), :],
                        sa_out.at[pl.ds(c * _GP + s * 80, 80), :])
        pltpu.sync_copy(spb.at[pl.ds(s * 80, 80), :], heb.at[pl.ds(0, 80), :])
        pltpu.sync_copy(heb.at[pl.ds(0, 80), :],
                        sb_out.at[pl.ds(c * _GP + s * 80, 80), :])

    @pl.when(s == 0)
    def _():
        pltpu.sync_copy(cnt_sp, zb1)
        pltpu.sync_copy(zb1, cnt_out.at[pl.ds(c * _GP, _GP)])


def _pool_sc(sa, sb, ss2, batchf):
    f = pl.kernel(
        _pool_body,
        out_type=[
            jax.ShapeDtypeStruct((2 * _GP, _HH), jnp.float32),
            jax.ShapeDtypeStruct((2 * _GP, _HH), jnp.float32),
            jax.ShapeDtypeStruct((2 * _GP,), jnp.float32),
        ],
        mesh=_mesh,
        compiler_params=_sc_params,
        scratch_types=[
            pltpu.VMEM((3072, _HH), jnp.float32),
            pltpu.VMEM((3072, _HH), jnp.float32),
            pltpu.VMEM((128, _HH), jnp.float32),
            pltpu.VMEM((128, _HH), jnp.float32),
            pltpu.VMEM((128,), jnp.int32),
            pltpu.VMEM((128,), jnp.float32),
            pltpu.VMEM((_GP,), jnp.float32),
            pltpu.VMEM((2, _H), jnp.float32),
            pltpu.VMEM_SHARED((_GP, _HH), jnp.float32),
            pltpu.VMEM_SHARED((_GP, _HH), jnp.float32),
            pltpu.VMEM_SHARED((_GP,), jnp.float32),
        ],
    )
    return f(sa, sb, ss2, batchf)


_BR = 2000
_NB = _N // _BR


def _prep_body(x_ref, p0_ref, p1_ref, w1_ref, b1_ref,
               dinv_ref, hsa_ref, hsb_ref, ia_ref, ib_ref):
    deg = p0_ref[...] + p1_ref[...] + 1.0
    dinv = lax.rsqrt(deg)
    h = jnp.dot(x_ref[...], w1_ref[...], preferred_element_type=jnp.float32)
    hs = h * dinv
    init = hs * dinv + b1_ref[...]
    dinv_ref[...] = dinv
    hsa_ref[...] = hs[:, :_HH]
    hsb_ref[...] = hs[:, _HH:]
    ia_ref[...] = init[:, :_HH]
    ib_ref[...] = init[:, _HH:]


def _prep_tc(x, p0, p1, W1, b1):
    return pl.pallas_call(
        _prep_body,
        grid=(_NB,),
        in_specs=[
            pl.BlockSpec((_BR, 13), lambda i: (i, 0)),
            pl.BlockSpec((_BR, 1), lambda i: (i, 0)),
            pl.BlockSpec((_BR, 1), lambda i: (i, 0)),
            pl.BlockSpec((13, _H), lambda i: (0, 0)),
            pl.BlockSpec((_H,), lambda i: (0,)),
        ],
        out_specs=[
            pl.BlockSpec((_BR, 1), lambda i: (i, 0)),
            pl.BlockSpec((_BR, _HH), lambda i: (i, 0)),
            pl.BlockSpec((_BR, _HH), lambda i: (i, 0)),
            pl.BlockSpec((_BR, _HH), lambda i: (i, 0)),
            pl.BlockSpec((_BR, _HH), lambda i: (i, 0)),
        ],
        out_shape=[
            jax.ShapeDtypeStruct((_N, 1), jnp.float32),
            jax.ShapeDtypeStruct((_N, _HH), jnp.float32),
            jax.ShapeDtypeStruct((_N, _HH), jnp.float32),
            jax.ShapeDtypeStruct((_N, _HH), jnp.float32),
            jax.ShapeDtypeStruct((_N, _HH), jnp.float32),
        ],
    )(x, p0, p1, W1, b1)


def _stats_body(xa_ref, xb_ref, g_ref, be_ref, o_ref, acc):
    i = pl.program_id(0)

    @pl.when(i == 0)
    def _():
        acc[...] = jnp.zeros_like(acc)

    xb = jnp.concatenate([xa_ref[...], xb_ref[...]], axis=1)
    acc[...] += jnp.stack([jnp.sum(xb, 0), jnp.sum(xb * xb, 0)])

    @pl.when(i == _NB - 1)
    def _():
        mu = acc[0, :] * (1.0 / _N)
        var = acc[1, :] * (1.0 / _N) - mu * mu
        sc = g_ref[...] * lax.rsqrt(var + 1e-5)
        o_ref[...] = jnp.stack([sc, be_ref[...] - mu * sc])


def _stats_tc(Sa, Sb, g, be):
    return pl.pallas_call(
        _stats_body,
        grid=(_NB,),
        in_specs=[
            pl.BlockSpec((_BR, _HH), lambda i: (i, 0)),
            pl.BlockSpec((_BR, _HH), lambda i: (i, 0)),
            pl.BlockSpec((_H,), lambda i: (0,)),
            pl.BlockSpec((_H,), lambda i: (0,)),
        ],
        out_specs=pl.BlockSpec((2, _H), lambda i: (0, 0)),
        out_shape=jax.ShapeDtypeStruct((2, _H), jnp.float32),
        scratch_shapes=[pltpu.VMEM((2, _H), jnp.float32)],
    )(Sa, Sb, g, be)


def _apply_body(sa_ref, sb_ref, ss_ref, dinv_ref, w2_ref, b2_ref,
                hsa_ref, hsb_ref, ia_ref, ib_ref):
    xb = jnp.concatenate([sa_ref[...], sb_ref[...]], axis=1)
    y = jnp.maximum(xb * ss_ref[0, :] + ss_ref[1, :], 0.0)
    hb = jnp.dot(y, w2_ref[...], preferred_element_type=jnp.float32)
    dinv = dinv_ref[...]
    hs = hb * dinv
    init = hs * dinv + b2_ref[...]
    hsa_ref[...] = hs[:, :_HH]
    hsb_ref[...] = hs[:, _HH:]
    ia_ref[...] = init[:, :_HH]
    ib_ref[...] = init[:, _HH:]


def _apply_tc(S1a, S1b, ss1, dinv, W2, b2):
    return pl.pallas_call(
        _apply_body,
        grid=(_NB,),
        in_specs=[
            pl.BlockSpec((_BR, _HH), lambda i: (i, 0)),
            pl.BlockSpec((_BR, _HH), lambda i: (i, 0)),
            pl.BlockSpec((2, _H), lambda i: (0, 0)),
            pl.BlockSpec((_BR, 1), lambda i: (i, 0)),
            pl.BlockSpec((_H, _H), lambda i: (0, 0)),
            pl.BlockSpec((_H,), lambda i: (0,)),
        ],
        out_specs=[
            pl.BlockSpec((_BR, _HH), lambda i: (i, 0)),
            pl.BlockSpec((_BR, _HH), lambda i: (i, 0)),
            pl.BlockSpec((_BR, _HH), lambda i: (i, 0)),
            pl.BlockSpec((_BR, _HH), lambda i: (i, 0)),
        ],
        out_shape=[
            jax.ShapeDtypeStruct((_N, _HH), jnp.float32),
            jax.ShapeDtypeStruct((_N, _HH), jnp.float32),
            jax.ShapeDtypeStruct((_N, _HH), jnp.float32),
            jax.ShapeDtypeStruct((_N, _HH), jnp.float32),
        ],
    )(S1a, S1b, ss1, dinv, W2, b2)


def _heads_body(sa_ref, sb_ref, cn_ref, gf_ref, wo_ref, bo_ref,
                wb_ref, bb_ref, o_ref, b_ref):
    sa = sa_ref[0, :_G, :] + sa_ref[1, :_G, :]
    sb = sb_ref[0, :_G, :] + sb_ref[1, :_G, :]
    cnt = jnp.maximum(cn_ref[0, :_G] + cn_ref[1, :_G], 1.0)[:, None]
    pa = sa / cnt
    pb = sb / cnt
    gf = gf_ref[...]
    o_ref[...] = jax.nn.sigmoid(
        pa @ wo_ref[:_HH, :] + pb @ wo_ref[_HH:_H, :]
        + gf @ wo_ref[_H:, :] + bo_ref[0])
    b_ref[...] = jax.nn.sigmoid(
        pa @ wb_ref[:_HH, :] + pb @ wb_ref[_HH:_H, :]
        + gf @ wb_ref[_H:, :] + bb_ref[0])


def _heads_tc(sa, sb, cnt, gf, Wo, bo, Wb, bb):
    return pl.pallas_call(
        _heads_body,
        out_shape=[
            jax.ShapeDtypeStruct((_G, 1), jnp.float32),
            jax.ShapeDtypeStruct((_G, 1), jnp.float32),
        ],
    )(sa, sb, cnt, gf, Wo, bo, Wb, bb)


def kernel(x, edge_index, edge_weight, batch, global_features,
           W1, b1, g1, be1, W2, b2, g2, be2, Wo, bo, Wb, bb):
    srcf = edge_index[0]
    dstf = edge_index[1]
    degp = _deg_sc(dstf.reshape(_R, 128), edge_weight.reshape(_R, 128))
    p0 = degp[:_N].reshape(_N, 1)
    p1 = degp[_NPAD:_NPAD + _N].reshape(_N, 1)

    dinv, hs1a, hs1b, i1a, i1b = _prep_tc(x, p0, p1, W1, b1)
    dinvf = jnp.pad(dinv[:, 0], (0, _NPAD - _N))

    pk = jnp.concatenate(
        [srcf.reshape(_R, 128), dstf.reshape(_R, 128),
         lax.bitcast_convert_type(edge_weight, jnp.int32).reshape(_R, 128)],
        axis=1).reshape(-1)

    S1a, S1b = _mp_kernel(hs1a, hs1b, i1a, i1b, pk, dinvf)
    ss1 = _stats_tc(S1a[:_N], S1b[:_N], g1, be1)
    hs2a, hs2b, i2a, i2b = _apply_tc(S1a[:_N], S1b[:_N], ss1, dinv, W2, b2)

    S2a, S2b = _mp_kernel(hs2a, hs2b, i2a, i2b, pk, dinvf)
    ss2 = _stats_tc(S2a[:_N], S2b[:_N], g2, be2)

    batchf = jnp.pad(batch, (0, _NPAD - _N), constant_values=_G)
    sa, sb, cnt = _pool_sc(S2a, S2b, ss2, batchf)
    orange, blue = _heads_tc(sa.reshape(2, _GP, _HH), sb.reshape(2, _GP, _HH),
                             cnt.reshape(2, _GP), global_features,
                             Wo, bo, Wb, bb)
    return (orange, blue)
